# baseline (speedup 1.0000x reference)
import functools

import jax
import jax.numpy as jnp
from jax import lax
from jax.experimental import pallas as pl
from jax.experimental.pallas import tpu as pltpu
from jax.experimental.pallas import tpu_sc as plsc

N, M, E = 10000, 5000, 320000
D = 128
NP = 10240
MP = 5120
C = 128
NW = 32

EP = 327680
EEP = 65536
VEP = 131072

_f32 = jnp.float32
_i32 = jnp.int32


def _mesh():
    return plsc.VectorSubcoreMesh(core_axis_name="c", subcore_axis_name="s")


def _zero_vmem(zb, rows, width):
    for r in range(rows):
        for k in range(width // 16):
            zb[r, pl.ds(k * 16, 16)] = jnp.zeros((16,), _f32)


def _zero_acc(acc, sid, rows_pt, zb, sem):
    nz = rows_pt // 16
    def fire(i, _):
        pltpu.async_copy(zb, acc.at[pl.ds(sid * rows_pt + i * 16, 16)], sem)
        return 0
    lax.fori_loop(0, nz, fire, 0)
    def drain(i, _):
        pltpu.make_async_copy(zb, acc.at[pl.ds(sid * rows_pt, 16)], sem).wait()
        return 0
    lax.fori_loop(0, nz, drain, 0)


def _copy_out(acc, out, cid, sid, rows_pt):
    pltpu.sync_copy(acc.at[pl.ds(sid * rows_pt, rows_pt)],
                    out.at[cid, pl.ds(sid * rows_pt, rows_pt)])


SB = 16


def _stream_phase(table, g2, a2, acc, gi, ai, r0, r1, sm0, sm1,
                  base, nsc, sb):
    def sc_body(s, _):
        row0 = base + s * sb
        pltpu.sync_copy(g2.at[pl.ds(row0, sb)], gi.at[pl.ds(0, sb)])
        pltpu.sync_copy(a2.at[pl.ds(row0, sb)], ai.at[pl.ds(0, sb)])
        pltpu.async_copy(table.at[gi.at[0]], r0, sm0)
        if sb > 1:
            pltpu.async_copy(table.at[gi.at[1]], r1, sm1)
        for j in range(sb):
            rb, sem = (r0, sm0) if j % 2 == 0 else (r1, sm1)
            pltpu.make_async_copy(table.at[gi.at[j]], rb, sem).wait()
            pltpu.sync_copy(rb, acc.at[ai.at[j]], add=True)
            if j + 2 < sb:
                pltpu.async_copy(table.at[gi.at[j + 2]], rb, sem)
        return 0
    lax.fori_loop(0, nsc, sc_body, 0)


def _make_inc(dirs):
    nc = EP // C // NW
    nsc = nc // SB
    nt = len(dirs)
    arows = NP if "s" in dirs else MP

    @functools.partial(
        pl.kernel, mesh=_mesh(),
        out_type=[jax.ShapeDtypeStruct((2, MP if d == "d" else NP, D), _f32)
                  for d in dirs],
        scratch_types=[
            pltpu.VMEM((SB, C), _i32), pltpu.VMEM((SB, C), _i32),
            pltpu.VMEM((C, D), _f32), pltpu.VMEM((C, D), _f32),
            pltpu.VMEM((16, D), _f32),
            pltpu.VMEM_SHARED((arows, D), _f32),
            pltpu.SemaphoreType.DMA, pltpu.SemaphoreType.DMA,
            pltpu.SemaphoreType.DMA,
        ],
    )
    def k(*args):
        tables = args[:nt]
        src, dst = args[nt], args[nt + 1]
        outs = args[nt + 2:nt + 2 + nt]
        gi, ai, r0, r1, zb, acc, sm0, sm1, smz = args[nt + 2 + nt:]
        cid = lax.axis_index("c")
        sid = lax.axis_index("s")
        wid = cid * 16 + sid
        _zero_vmem(zb, 16, D)

        for p, dr in enumerate(dirs):
            rpt = (MP if dr == "d" else NP) // 16
            _zero_acc(acc, sid, rpt, zb, smz)
            plsc.subcore_barrier()

            table = tables[p]
            g2, a2 = (src, dst) if dr == "d" else (dst, src)
            _stream_phase(table, g2, a2, acc, gi, ai, r0, r1, sm0, sm1,
                          wid * nc, nsc, SB)

            plsc.subcore_barrier()
            _copy_out(acc, outs[p], cid, sid, rpt)
            plsc.subcore_barrier()

    return k


SBC = 16


def _make_coo_dual():
    @functools.partial(
        pl.kernel, mesh=_mesh(),
        out_type=[jax.ShapeDtypeStruct((2, MP, D), _f32),
                  jax.ShapeDtypeStruct((2, NP, D), _f32)],
        scratch_types=[
            pltpu.VMEM((SBC, C), _i32), pltpu.VMEM((SBC, C), _i32),
            pltpu.VMEM((SBC, C), _f32),
            pltpu.VMEM((C, D), _f32), pltpu.VMEM((C, D), _f32),
            pltpu.VMEM((16, D), _f32),
            pltpu.VMEM_SHARED((NP, D), _f32),
            pltpu.SemaphoreType.DMA, pltpu.SemaphoreType.DMA,
            pltpu.SemaphoreType.DMA,
        ],
    )
    def k(tA, tB, ecol, erow, evals, vcol, vrow, vvals, oA, oB,
          gi, ai, wv, r0, r1, zb, acc, sm0, sm1, smz):
        cid = lax.axis_index("c")
        sid = lax.axis_index("s")
        wid = cid * 16 + sid
        _zero_vmem(zb, 16, D)

        def scale_rows(rows, j):
            def sbody(g, _):
                v = wv[j, pl.ds(g * 16, 16)]
                for l in range(16):
                    b = lax.broadcast(v[l], (16,))
                    for kk in range(D // 16):
                        rows[g * 16 + l, pl.ds(kk * 16, 16)] = (
                            rows[g * 16 + l, pl.ds(kk * 16, 16)] * b)
                return 0
            lax.fori_loop(0, C // 16, sbody, 0)

        def run_list(col, row, vals, table, out, tot, rpt, sb):
            nc = tot // C // NW
            nsc = nc // sb
            _zero_acc(acc, sid, rpt, zb, smz)
            plsc.subcore_barrier()

            def sc_body(s, _):
                row0 = wid * nc + s * sb
                pltpu.sync_copy(col.at[pl.ds(row0, sb)], gi.at[pl.ds(0, sb)])
                pltpu.sync_copy(row.at[pl.ds(row0, sb)], ai.at[pl.ds(0, sb)])
                pltpu.sync_copy(vals.at[pl.ds(row0, sb)], wv.at[pl.ds(0, sb)])
                pltpu.async_copy(table.at[gi.at[0]], r0, sm0)
                if sb > 1:
                    pltpu.async_copy(table.at[gi.at[1]], r1, sm1)
                for j in range(sb):
                    rb, sem = (r0, sm0) if j % 2 == 0 else (r1, sm1)
                    pltpu.make_async_copy(table.at[gi.at[j]], rb, sem).wait()
                    scale_rows(rb, j)
                    pltpu.sync_copy(rb, acc.at[ai.at[j]], add=True)
                    if j + 2 < sb:
                        pltpu.async_copy(table.at[gi.at[j + 2]], rb, sem)
                return 0
            lax.fori_loop(0, nsc, sc_body, 0)

            plsc.subcore_barrier()
            _copy_out(acc, out, cid, sid, rpt)
            plsc.subcore_barrier()

        run_list(ecol, erow, evals, tA, oA, EEP, MP // 16, 8)
        run_list(vcol, vrow, vvals, tB, oB, VEP, NP // 16, 16)

    return k


def _mmT(x, w):
    return lax.dot_general(x, w, (((1,), (1,)), ((), ())),
                           preferred_element_type=_f32)


def _tc(body, grid, in_specs, out_specs, out_shape):
    return pl.pallas_call(body, grid=grid, in_specs=in_specs,
                          out_specs=out_specs, out_shape=out_shape)


RB = 512


def _rows_spec(w, nd=2):
    if nd == 2:
        return pl.BlockSpec((RB, w), lambda i: (i, 0))
    return pl.BlockSpec((2, RB, w), lambda i: (0, i, 0))


def _full_spec(shape):
    n = len(shape)
    return pl.BlockSpec(shape, lambda i: (0,) * n)


def _t0_body(vf_ref, inv_ref, o_ref):
    o_ref[...] = vf_ref[...] * inv_ref[...]


def _t1_body(o1_ref, cd_ref, ef_ref, wa_ref, wb_ref, b1_ref, a_ref):
    sv = o1_ref[0] + o1_ref[1]
    cc = (cd_ref[0] + cd_ref[1])[:, 0:1]
    efb = _mmT(ef_ref[...], wb_ref[...]) + b1_ref[...]
    a_ref[...] = _mmT(sv, wa_ref[...]) + cc * efb


def _sum2_body(x_ref, o_ref):
    o_ref[...] = x_ref[0] + x_ref[1]


def _sum2p_body(x_ref, y_ref, o_ref):
    o_ref[...] = x_ref[0] + x_ref[1] + y_ref[...]


def _relu_mm_body(x_ref, w_ref, o_ref):
    s = x_ref[0] + x_ref[1]
    o_ref[...] = jnp.maximum(_mmT(s, w_ref[...]), 0.0)


def _t4_body(sv2_ref, cd_ref, ef_ref, inv_ref, e2_ref,
             w2a_ref, w2b_ref, b2_ref, we_ref, o_ref):
    deg = (cd_ref[0] + cd_ref[1])[:, 1:2]
    sv2 = sv2_ref[0] + sv2_ref[1]
    efb = _mmT(ef_ref[...], w2b_ref[...]) + b2_ref[...]
    bb = inv_ref[...] * (_mmT(sv2, w2a_ref[...]) + deg * efb)
    o_ref[...] = jnp.maximum(_mmT(e2_ref[...] + bb, we_ref[...]), 0.0)


_k1 = _make_inc(("d", "d", "s"))
_inc_dual = _make_inc(("d", "s"))
_inc_single = _make_inc(("d",))
_coo_dual = _make_coo_dual()


def kernel(vfeat, efeat, invDV, invDE, in_src, in_dst, vmat_indices,
           vmat_values, emat_indices, emat_values, W_v, W_e,
           psi1_w, psi1_b, psi2_w, psi2_b):
    vfeat_p = jnp.pad(vfeat, ((0, NP - N), (0, 0)))
    efeat_p = jnp.pad(efeat, ((0, MP - M), (0, 0)))
    invDV_p = jnp.pad(invDV, (0, NP - N))
    invDE_p = jnp.pad(invDE, (0, MP - M)).reshape(MP, 1)
    src_p = jnp.concatenate([in_src, jnp.full((EP - E,), N, _i32)]).reshape(EP // C, C)
    dst_p = jnp.concatenate([in_dst, jnp.full((EP - E,), M, _i32)]).reshape(EP // C, C)
    erow_p = jnp.concatenate([emat_indices[0], jnp.full((EEP - 50000,), M, _i32)]).reshape(EEP // C, C)
    ecol_p = jnp.concatenate([emat_indices[1], jnp.full((EEP - 50000,), M, _i32)]).reshape(EEP // C, C)
    eval_p = jnp.concatenate([emat_values, jnp.zeros((EEP - 50000,), _f32)]).reshape(EEP // C, C)
    vrow_p = jnp.concatenate([vmat_indices[0], jnp.full((VEP - 100000,), N, _i32)]).reshape(VEP // C, C)
    vcol_p = jnp.concatenate([vmat_indices[1], jnp.full((VEP - 100000,), N, _i32)]).reshape(VEP // C, C)
    vval_p = jnp.concatenate([vmat_values, jnp.zeros((VEP - 100000,), _f32)]).reshape(VEP // C, C)
    wa, wb = psi1_w[:, :D], psi1_w[:, D:]
    w2a, w2b = psi2_w[:, :D], psi2_w[:, D:]
    b1 = psi1_b.reshape(1, D)
    b2 = psi2_b.reshape(1, D)
    mask = jnp.pad(jnp.ones((N,), _f32), (0, NP - N))
    t_cd = jnp.pad(jnp.stack([invDV_p, mask], axis=1), ((0, 0), (0, D - 2)))

    gm = MP // RB
    gn = NP // RB

    t1_table = _tc(_t0_body, (gn,),
                   [_rows_spec(D), _rows_spec(1)],
                   _rows_spec(D),
                   jax.ShapeDtypeStruct((NP, D), _f32))(
        vfeat_p, invDV_p.reshape(NP, 1))

    o1, ocd, o2 = _k1(t1_table, t_cd, efeat_p, src_p, dst_p)

    a_p = _tc(_t1_body, (gm,),
              [_rows_spec(D, 3), _rows_spec(D, 3), _rows_spec(D),
               _full_spec((D, D)), _full_spec((D, D)), _full_spec((1, D))],
              _rows_spec(D),
              jax.ShapeDtypeStruct((MP, D), _f32))(o1, ocd, efeat_p, wa, wb, b1)
    vf2pre = _tc(_sum2_body, (gn,), [_rows_spec(D, 3)], _rows_spec(D),
                 jax.ShapeDtypeStruct((NP, D), _f32))(o2)

    a2p, vf2p = _coo_dual(a_p, vf2pre, ecol_p, erow_p, eval_p,
                          vcol_p, vrow_p, vval_p)

    efeat_mid = _tc(_sum2p_body, (gm,), [_rows_spec(D, 3), _rows_spec(D)],
                    _rows_spec(D),
                    jax.ShapeDtypeStruct((MP, D), _f32))(a2p, efeat_p)
    vf2 = _tc(_sum2_body, (gn,), [_rows_spec(D, 3)], _rows_spec(D),
              jax.ShapeDtypeStruct((NP, D), _f32))(vf2p)

    e2p, vfp = _inc_dual(vf2, efeat_mid, src_p, dst_p)

    vfo_p = _tc(_relu_mm_body, (gn,), [_rows_spec(D, 3), _full_spec((D, D))],
                _rows_spec(D),
                jax.ShapeDtypeStruct((NP, D), _f32))(vfp, W_v)
    e2 = _tc(_sum2_body, (gm,), [_rows_spec(D, 3)], _rows_spec(D),
             jax.ShapeDtypeStruct((MP, D), _f32))(e2p)

    (sv2p,) = _inc_single(vfo_p, src_p, dst_p)

    efo_p = _tc(_t4_body, (gm,),
                [_rows_spec(D, 3), _rows_spec(D, 3), _rows_spec(D),
                 _rows_spec(1), _rows_spec(D), _full_spec((D, D)),
                 _full_spec((D, D)), _full_spec((1, D)), _full_spec((D, D))],
                _rows_spec(D),
                jax.ShapeDtypeStruct((MP, D), _f32))(
        sv2p, ocd, efeat_p, invDE_p, e2, w2a, w2b, b2, W_e)

    return (vfo_p[:N], efo_p[:M])


# speedup vs baseline: 1.6197x; 1.6197x over previous
import functools

import jax
import jax.numpy as jnp
from jax import lax
from jax.experimental import pallas as pl
from jax.experimental.pallas import tpu as pltpu
from jax.experimental.pallas import tpu_sc as plsc

N, M, E = 10000, 5000, 320000
D = 128
NP = 10240
MP = 5120
C = 128
NW = 32

EP = 323584
EEP = 53248
VEP = 106496

_f32 = jnp.float32
_i32 = jnp.int32


def _mesh():
    return plsc.VectorSubcoreMesh(core_axis_name="c", subcore_axis_name="s")


def _zero_vmem(zb, rows, width):
    for r in range(rows):
        for k in range(width // 16):
            zb[r, pl.ds(k * 16, 16)] = jnp.zeros((16,), _f32)


def _zero_acc(acc, sid, rows_pt, zb):
    def body(i, _):
        pltpu.sync_copy(zb, acc.at[pl.ds(sid * rows_pt + i * 16, 16)])
        return 0
    lax.fori_loop(0, rows_pt // 16, body, 0)


def _copy_out(acc, out, cid, sid, rows_pt):
    pltpu.sync_copy(acc.at[pl.ds(sid * rows_pt, rows_pt)],
                    out.at[cid, pl.ds(sid * rows_pt, rows_pt)])


def _make_inc(dirs):
    pw = EP // NW
    nchunk = pw // C
    nt = len(dirs)
    arows = NP if "s" in dirs else MP

    @functools.partial(
        pl.kernel, mesh=_mesh(),
        out_type=[jax.ShapeDtypeStruct((2, MP if d == "d" else NP, D), _f32)
                  for d in dirs],
        scratch_types=[
            pltpu.VMEM((C,), _i32), pltpu.VMEM((C,), _i32),
            pltpu.VMEM((C, D), _f32),
            pltpu.VMEM((16, D), _f32),
            pltpu.VMEM_SHARED((arows, D), _f32),
            pltpu.SemaphoreType.DMA,
        ],
    )
    def k(*args):
        tables = args[:nt]
        src, dst = args[nt], args[nt + 1]
        outs = args[nt + 2:nt + 2 + nt]
        s_idx, d_idx, rows, zb, acc, sem = args[nt + 2 + nt:]
        cid = lax.axis_index("c")
        sid = lax.axis_index("s")
        wid = cid * 16 + sid
        _zero_vmem(zb, 16, D)

        for p, dr in enumerate(dirs):
            rpt = (MP if dr == "d" else NP) // 16
            _zero_acc(acc, sid, rpt, zb)
            plsc.subcore_barrier()

            table = tables[p]
            gl, al = (src, dst) if dr == "d" else (dst, src)

            def body(i, _, table=table, gl=gl, al=al):
                base = wid * pw + i * C
                pltpu.sync_copy(gl.at[pl.ds(base, C)], s_idx)
                pltpu.sync_copy(al.at[pl.ds(base, C)], d_idx)
                pltpu.async_copy(table.at[s_idx], rows, sem).wait()
                pltpu.sync_copy(rows, acc.at[d_idx], add=True)
                return 0
            lax.fori_loop(0, nchunk, body, 0)

            plsc.subcore_barrier()
            _copy_out(acc, outs[p], cid, sid, rpt)
            plsc.subcore_barrier()

    return k


def _make_coo_dual():
    @functools.partial(
        pl.kernel, mesh=_mesh(),
        out_type=[jax.ShapeDtypeStruct((2, MP, D), _f32),
                  jax.ShapeDtypeStruct((2, NP, D), _f32)],
        scratch_types=[
            pltpu.VMEM((C,), _i32), pltpu.VMEM((C,), _i32),
            pltpu.VMEM((C,), _f32),
            pltpu.VMEM((C, D), _f32),
            pltpu.VMEM((16, D), _f32),
            pltpu.VMEM_SHARED((NP, D), _f32),
            pltpu.SemaphoreType.DMA,
        ],
    )
    def k(tA, tB, ecol, erow, evals, vcol, vrow, vvals, oA, oB,
          g0, a0, w0, r0, zb, acc, sem):
        cid = lax.axis_index("c")
        sid = lax.axis_index("s")
        wid = cid * 16 + sid
        _zero_vmem(zb, 16, D)

        def scale_rows(rows, wv):
            def sbody(g, _):
                v = wv[pl.ds(g * 16, 16)]
                for l in range(16):
                    b = lax.broadcast(v[l], (16,))
                    for kk in range(D // 16):
                        rows[g * 16 + l, pl.ds(kk * 16, 16)] = (
                            rows[g * 16 + l, pl.ds(kk * 16, 16)] * b)
                return 0
            lax.fori_loop(0, C // 16, sbody, 0)

        def run_list(col, row, vals, table, out, tot, rpt):
            pw = tot // NW
            nh = pw // C
            _zero_acc(acc, sid, rpt, zb)
            plsc.subcore_barrier()

            def body(i, _):
                b = wid * pw + i * C
                pltpu.sync_copy(col.at[pl.ds(b, C)], g0)
                pltpu.sync_copy(row.at[pl.ds(b, C)], a0)
                pltpu.sync_copy(vals.at[pl.ds(b, C)], w0)
                pltpu.async_copy(table.at[g0], r0, sem).wait()
                scale_rows(r0, w0)
                pltpu.sync_copy(r0, acc.at[a0], add=True)
                return 0
            lax.fori_loop(0, nh, body, 0)

            plsc.subcore_barrier()
            _copy_out(acc, out, cid, sid, rpt)
            plsc.subcore_barrier()

        run_list(ecol, erow, evals, tA, oA, EEP, MP // 16)
        run_list(vcol, vrow, vvals, tB, oB, VEP, NP // 16)

    return k


def _mmT(x, w):
    return lax.dot_general(x, w, (((1,), (1,)), ((), ())),
                           preferred_element_type=_f32)


def _tc(body, grid, in_specs, out_specs, out_shape):
    return pl.pallas_call(body, grid=grid, in_specs=in_specs,
                          out_specs=out_specs, out_shape=out_shape)


RB = 512


def _rows_spec(w, nd=2):
    if nd == 2:
        return pl.BlockSpec((RB, w), lambda i: (i, 0))
    return pl.BlockSpec((2, RB, w), lambda i: (0, i, 0))


def _full_spec(shape):
    n = len(shape)
    return pl.BlockSpec(shape, lambda i: (0,) * n)


def _t0_body(vf_ref, inv_ref, o_ref):
    o_ref[...] = vf_ref[...] * inv_ref[...]


def _t1_body(o1_ref, cd_ref, ef_ref, wa_ref, wb_ref, b1_ref, a_ref):
    sv = o1_ref[0] + o1_ref[1]
    cc = (cd_ref[0] + cd_ref[1])[:, 0:1]
    efb = _mmT(ef_ref[...], wb_ref[...]) + b1_ref[...]
    a_ref[...] = _mmT(sv, wa_ref[...]) + cc * efb


def _sum2_body(x_ref, o_ref):
    o_ref[...] = x_ref[0] + x_ref[1]


def _sum2p_body(x_ref, y_ref, o_ref):
    o_ref[...] = x_ref[0] + x_ref[1] + y_ref[...]


def _relu_mm_body(x_ref, w_ref, o_ref):
    s = x_ref[0] + x_ref[1]
    o_ref[...] = jnp.maximum(_mmT(s, w_ref[...]), 0.0)


def _t4_body(sv2_ref, cd_ref, ef_ref, inv_ref, e2_ref,
             w2a_ref, w2b_ref, b2_ref, we_ref, o_ref):
    deg = (cd_ref[0] + cd_ref[1])[:, 1:2]
    sv2 = sv2_ref[0] + sv2_ref[1]
    efb = _mmT(ef_ref[...], w2b_ref[...]) + b2_ref[...]
    bb = inv_ref[...] * (_mmT(sv2, w2a_ref[...]) + deg * efb)
    o_ref[...] = jnp.maximum(_mmT(e2_ref[...] + bb, we_ref[...]), 0.0)


_k1 = _make_inc(("d", "d", "s"))
_inc_dual = _make_inc(("d", "s"))
_inc_single = _make_inc(("d",))
_coo_dual = _make_coo_dual()


def kernel(vfeat, efeat, invDV, invDE, in_src, in_dst, vmat_indices,
           vmat_values, emat_indices, emat_values, W_v, W_e,
           psi1_w, psi1_b, psi2_w, psi2_b):
    vfeat_p = jnp.pad(vfeat, ((0, NP - N), (0, 0)))
    efeat_p = jnp.pad(efeat, ((0, MP - M), (0, 0)))
    invDV_p = jnp.pad(invDV, (0, NP - N))
    invDE_p = jnp.pad(invDE, (0, MP - M)).reshape(MP, 1)
    src_p = jnp.concatenate([in_src, jnp.full((EP - E,), N, _i32)])
    dst_p = jnp.concatenate([in_dst, jnp.full((EP - E,), M, _i32)])
    erow_p = jnp.concatenate([emat_indices[0], jnp.full((EEP - 50000,), M, _i32)])
    ecol_p = jnp.concatenate([emat_indices[1], jnp.full((EEP - 50000,), M, _i32)])
    eval_p = jnp.concatenate([emat_values, jnp.zeros((EEP - 50000,), _f32)])
    vrow_p = jnp.concatenate([vmat_indices[0], jnp.full((VEP - 100000,), N, _i32)])
    vcol_p = jnp.concatenate([vmat_indices[1], jnp.full((VEP - 100000,), N, _i32)])
    vval_p = jnp.concatenate([vmat_values, jnp.zeros((VEP - 100000,), _f32)])
    wa, wb = psi1_w[:, :D], psi1_w[:, D:]
    w2a, w2b = psi2_w[:, :D], psi2_w[:, D:]
    b1 = psi1_b.reshape(1, D)
    b2 = psi2_b.reshape(1, D)
    mask = jnp.pad(jnp.ones((N,), _f32), (0, NP - N))
    t_cd = jnp.pad(jnp.stack([invDV_p, mask], axis=1), ((0, 0), (0, D - 2)))

    gm = MP // RB
    gn = NP // RB

    t1_table = _tc(_t0_body, (gn,),
                   [_rows_spec(D), _rows_spec(1)],
                   _rows_spec(D),
                   jax.ShapeDtypeStruct((NP, D), _f32))(
        vfeat_p, invDV_p.reshape(NP, 1))

    o1, ocd, o2 = _k1(t1_table, t_cd, efeat_p, src_p, dst_p)

    a_p = _tc(_t1_body, (gm,),
              [_rows_spec(D, 3), _rows_spec(D, 3), _rows_spec(D),
               _full_spec((D, D)), _full_spec((D, D)), _full_spec((1, D))],
              _rows_spec(D),
              jax.ShapeDtypeStruct((MP, D), _f32))(o1, ocd, efeat_p, wa, wb, b1)
    vf2pre = _tc(_sum2_body, (gn,), [_rows_spec(D, 3)], _rows_spec(D),
                 jax.ShapeDtypeStruct((NP, D), _f32))(o2)

    a2p, vf2p = _coo_dual(a_p, vf2pre, ecol_p, erow_p, eval_p,
                          vcol_p, vrow_p, vval_p)

    efeat_mid = _tc(_sum2p_body, (gm,), [_rows_spec(D, 3), _rows_spec(D)],
                    _rows_spec(D),
                    jax.ShapeDtypeStruct((MP, D), _f32))(a2p, efeat_p)
    vf2 = _tc(_sum2_body, (gn,), [_rows_spec(D, 3)], _rows_spec(D),
              jax.ShapeDtypeStruct((NP, D), _f32))(vf2p)

    e2p, vfp = _inc_dual(vf2, efeat_mid, src_p, dst_p)

    vfo_p = _tc(_relu_mm_body, (gn,), [_rows_spec(D, 3), _full_spec((D, D))],
                _rows_spec(D),
                jax.ShapeDtypeStruct((NP, D), _f32))(vfp, W_v)
    e2 = _tc(_sum2_body, (gm,), [_rows_spec(D, 3)], _rows_spec(D),
             jax.ShapeDtypeStruct((MP, D), _f32))(e2p)

    (sv2p,) = _inc_single(vfo_p, src_p, dst_p)

    efo_p = _tc(_t4_body, (gm,),
                [_rows_spec(D, 3), _rows_spec(D, 3), _rows_spec(D),
                 _rows_spec(1), _rows_spec(D), _full_spec((D, D)),
                 _full_spec((D, D)), _full_spec((1, D)), _full_spec((D, D))],
                _rows_spec(D),
                jax.ShapeDtypeStruct((MP, D), _f32))(
        sv2p, ocd, efeat_p, invDE_p, e2, w2a, w2b, b2, W_e)

    return (vfo_p[:N], efo_p[:M])


# speedup vs baseline: 1.7439x; 1.0767x over previous
import functools

import jax
import jax.numpy as jnp
from jax import lax
from jax.experimental import pallas as pl
from jax.experimental.pallas import tpu as pltpu
from jax.experimental.pallas import tpu_sc as plsc

N, M, E = 10000, 5000, 320000
D = 128
NP = 10240
MP = 5120
C = 128
NW = 32

EP = 323584
EEP = 53248
VEP = 106496

_f32 = jnp.float32
_i32 = jnp.int32


def _mesh():
    return plsc.VectorSubcoreMesh(core_axis_name="c", subcore_axis_name="s")


def _zero_vmem(zb, rows, width):
    for r in range(rows):
        for k in range(width // 16):
            zb[r, pl.ds(k * 16, 16)] = jnp.zeros((16,), _f32)


def _zero_acc(acc, sid, rows_pt, zb):
    def body(i, _):
        pltpu.sync_copy(zb, acc.at[pl.ds(sid * rows_pt + i * 16, 16)])
        return 0
    lax.fori_loop(0, rows_pt // 16, body, 0)


def _copy_out(acc, out, cid, sid, rows_pt):
    pltpu.sync_copy(acc.at[pl.ds(sid * rows_pt, rows_pt)],
                    out.at[cid, pl.ds(sid * rows_pt, rows_pt)])


def _make_inc(dirs):
    pw = EP // NW
    nchunk = pw // C
    nt = len(dirs)
    arows = NP if "s" in dirs else MP

    @functools.partial(
        pl.kernel, mesh=_mesh(),
        out_type=[jax.ShapeDtypeStruct((2, MP if d == "d" else NP, D), _f32)
                  for d in dirs],
        scratch_types=[
            pltpu.VMEM((C,), _i32), pltpu.VMEM((C,), _i32),
            pltpu.VMEM((C, D), _f32),
            pltpu.VMEM((16, D), _f32),
            pltpu.VMEM_SHARED((arows, D), _f32),
            pltpu.SemaphoreType.DMA, pltpu.SemaphoreType.DMA,
            pltpu.SemaphoreType.DMA, pltpu.SemaphoreType.DMA,
        ],
    )
    def k(*args):
        tables = args[:nt]
        src, dst = args[nt], args[nt + 1]
        outs = args[nt + 2:nt + 2 + nt]
        s_idx, d_idx, rows, zb, acc, sem, sma, smb, smw = args[nt + 2 + nt:]
        cid = lax.axis_index("c")
        sid = lax.axis_index("s")
        wid = cid * 16 + sid
        _zero_vmem(zb, 16, D)

        for p, dr in enumerate(dirs):
            rpt = (MP if dr == "d" else NP) // 16
            _zero_acc(acc, sid, rpt, zb)
            plsc.subcore_barrier()

            table = tables[p]
            gl, al = (src, dst) if dr == "d" else (dst, src)

            def body(i, _, table=table, gl=gl, al=al):
                base = wid * pw + i * C
                ca = pltpu.async_copy(gl.at[pl.ds(base, C)], s_idx, sma)
                cb = pltpu.async_copy(al.at[pl.ds(base, C)], d_idx, smb)
                ca.wait()
                cg = pltpu.async_copy(table.at[s_idx], rows, sem)
                cb.wait()
                cg.wait()
                pltpu.async_copy(rows, acc.at[d_idx], smw, add=True).wait()
                return 0
            lax.fori_loop(0, nchunk, body, 0)

            plsc.subcore_barrier()
            _copy_out(acc, outs[p], cid, sid, rpt)
            plsc.subcore_barrier()

    return k


def _make_coo_dual():
    @functools.partial(
        pl.kernel, mesh=_mesh(),
        out_type=[jax.ShapeDtypeStruct((2, MP, D), _f32),
                  jax.ShapeDtypeStruct((2, NP, D), _f32)],
        scratch_types=[
            pltpu.VMEM((C,), _i32), pltpu.VMEM((C,), _i32),
            pltpu.VMEM((C,), _f32),
            pltpu.VMEM((C, D), _f32),
            pltpu.VMEM((16, D), _f32),
            pltpu.VMEM_SHARED((NP, D), _f32),
            pltpu.SemaphoreType.DMA,
        ],
    )
    def k(tA, tB, ecol, erow, evals, vcol, vrow, vvals, oA, oB,
          g0, a0, w0, r0, zb, acc, sem):
        cid = lax.axis_index("c")
        sid = lax.axis_index("s")
        wid = cid * 16 + sid
        _zero_vmem(zb, 16, D)

        def scale_rows(rows, wv):
            def sbody(g, _):
                v = wv[pl.ds(g * 16, 16)]
                for l in range(16):
                    b = lax.broadcast(v[l], (16,))
                    for kk in range(D // 16):
                        rows[g * 16 + l, pl.ds(kk * 16, 16)] = (
                            rows[g * 16 + l, pl.ds(kk * 16, 16)] * b)
                return 0
            lax.fori_loop(0, C // 16, sbody, 0)

        def run_list(col, row, vals, table, out, tot, rpt):
            pw = tot // NW
            nh = pw // C
            _zero_acc(acc, sid, rpt, zb)
            plsc.subcore_barrier()

            def body(i, _):
                b = wid * pw + i * C
                pltpu.sync_copy(col.at[pl.ds(b, C)], g0)
                pltpu.sync_copy(row.at[pl.ds(b, C)], a0)
                pltpu.sync_copy(vals.at[pl.ds(b, C)], w0)
                pltpu.async_copy(table.at[g0], r0, sem).wait()
                scale_rows(r0, w0)
                pltpu.sync_copy(r0, acc.at[a0], add=True)
                return 0
            lax.fori_loop(0, nh, body, 0)

            plsc.subcore_barrier()
            _copy_out(acc, out, cid, sid, rpt)
            plsc.subcore_barrier()

        run_list(ecol, erow, evals, tA, oA, EEP, MP // 16)
        run_list(vcol, vrow, vvals, tB, oB, VEP, NP // 16)

    return k


def _mmT(x, w):
    return lax.dot_general(x, w, (((1,), (1,)), ((), ())),
                           preferred_element_type=_f32)


def _tc(body, grid, in_specs, out_specs, out_shape):
    return pl.pallas_call(body, grid=grid, in_specs=in_specs,
                          out_specs=out_specs, out_shape=out_shape)


RB = 512


def _rows_spec(w, nd=2):
    if nd == 2:
        return pl.BlockSpec((RB, w), lambda i: (i, 0))
    return pl.BlockSpec((2, RB, w), lambda i: (0, i, 0))


def _full_spec(shape):
    n = len(shape)
    return pl.BlockSpec(shape, lambda i: (0,) * n)


def _t0_body(vf_ref, inv_ref, o_ref):
    o_ref[...] = vf_ref[...] * inv_ref[...]


def _t1_body(o1_ref, cd_ref, ef_ref, wa_ref, wb_ref, b1_ref, a_ref):
    sv = o1_ref[0] + o1_ref[1]
    cc = (cd_ref[0] + cd_ref[1])[:, 0:1]
    efb = _mmT(ef_ref[...], wb_ref[...]) + b1_ref[...]
    a_ref[...] = _mmT(sv, wa_ref[...]) + cc * efb


def _sum2_body(x_ref, o_ref):
    o_ref[...] = x_ref[0] + x_ref[1]


def _sum2p_body(x_ref, y_ref, o_ref):
    o_ref[...] = x_ref[0] + x_ref[1] + y_ref[...]


def _relu_mm_body(x_ref, w_ref, o_ref):
    s = x_ref[0] + x_ref[1]
    o_ref[...] = jnp.maximum(_mmT(s, w_ref[...]), 0.0)


def _t4_body(sv2_ref, cd_ref, ef_ref, inv_ref, e2_ref,
             w2a_ref, w2b_ref, b2_ref, we_ref, o_ref):
    deg = (cd_ref[0] + cd_ref[1])[:, 1:2]
    sv2 = sv2_ref[0] + sv2_ref[1]
    efb = _mmT(ef_ref[...], w2b_ref[...]) + b2_ref[...]
    bb = inv_ref[...] * (_mmT(sv2, w2a_ref[...]) + deg * efb)
    o_ref[...] = jnp.maximum(_mmT(e2_ref[...] + bb, we_ref[...]), 0.0)


_k1 = _make_inc(("d", "d", "s"))
_inc_dual = _make_inc(("d", "s"))
_inc_single = _make_inc(("d",))
_coo_dual = _make_coo_dual()


def kernel(vfeat, efeat, invDV, invDE, in_src, in_dst, vmat_indices,
           vmat_values, emat_indices, emat_values, W_v, W_e,
           psi1_w, psi1_b, psi2_w, psi2_b):
    vfeat_p = jnp.pad(vfeat, ((0, NP - N), (0, 0)))
    efeat_p = jnp.pad(efeat, ((0, MP - M), (0, 0)))
    invDV_p = jnp.pad(invDV, (0, NP - N))
    invDE_p = jnp.pad(invDE, (0, MP - M)).reshape(MP, 1)
    src_p = jnp.concatenate([in_src, jnp.full((EP - E,), N, _i32)])
    dst_p = jnp.concatenate([in_dst, jnp.full((EP - E,), M, _i32)])
    erow_p = jnp.concatenate([emat_indices[0], jnp.full((EEP - 50000,), M, _i32)])
    ecol_p = jnp.concatenate([emat_indices[1], jnp.full((EEP - 50000,), M, _i32)])
    eval_p = jnp.concatenate([emat_values, jnp.zeros((EEP - 50000,), _f32)])
    vrow_p = jnp.concatenate([vmat_indices[0], jnp.full((VEP - 100000,), N, _i32)])
    vcol_p = jnp.concatenate([vmat_indices[1], jnp.full((VEP - 100000,), N, _i32)])
    vval_p = jnp.concatenate([vmat_values, jnp.zeros((VEP - 100000,), _f32)])
    wa, wb = psi1_w[:, :D], psi1_w[:, D:]
    w2a, w2b = psi2_w[:, :D], psi2_w[:, D:]
    b1 = psi1_b.reshape(1, D)
    b2 = psi2_b.reshape(1, D)
    mask = jnp.pad(jnp.ones((N,), _f32), (0, NP - N))
    t_cd = jnp.pad(jnp.stack([invDV_p, mask], axis=1), ((0, 0), (0, D - 2)))

    gm = MP // RB
    gn = NP // RB

    t1_table = _tc(_t0_body, (gn,),
                   [_rows_spec(D), _rows_spec(1)],
                   _rows_spec(D),
                   jax.ShapeDtypeStruct((NP, D), _f32))(
        vfeat_p, invDV_p.reshape(NP, 1))

    o1, ocd, o2 = _k1(t1_table, t_cd, efeat_p, src_p, dst_p)

    a_p = _tc(_t1_body, (gm,),
              [_rows_spec(D, 3), _rows_spec(D, 3), _rows_spec(D),
               _full_spec((D, D)), _full_spec((D, D)), _full_spec((1, D))],
              _rows_spec(D),
              jax.ShapeDtypeStruct((MP, D), _f32))(o1, ocd, efeat_p, wa, wb, b1)
    vf2pre = _tc(_sum2_body, (gn,), [_rows_spec(D, 3)], _rows_spec(D),
                 jax.ShapeDtypeStruct((NP, D), _f32))(o2)

    a2p, vf2p = _coo_dual(a_p, vf2pre, ecol_p, erow_p, eval_p,
                          vcol_p, vrow_p, vval_p)

    efeat_mid = _tc(_sum2p_body, (gm,), [_rows_spec(D, 3), _rows_spec(D)],
                    _rows_spec(D),
                    jax.ShapeDtypeStruct((MP, D), _f32))(a2p, efeat_p)
    vf2 = _tc(_sum2_body, (gn,), [_rows_spec(D, 3)], _rows_spec(D),
              jax.ShapeDtypeStruct((NP, D), _f32))(vf2p)

    e2p, vfp = _inc_dual(vf2, efeat_mid, src_p, dst_p)

    vfo_p = _tc(_relu_mm_body, (gn,), [_rows_spec(D, 3), _full_spec((D, D))],
                _rows_spec(D),
                jax.ShapeDtypeStruct((NP, D), _f32))(vfp, W_v)
    e2 = _tc(_sum2_body, (gm,), [_rows_spec(D, 3)], _rows_spec(D),
             jax.ShapeDtypeStruct((MP, D), _f32))(e2p)

    (sv2p,) = _inc_single(vfo_p, src_p, dst_p)

    efo_p = _tc(_t4_body, (gm,),
                [_rows_spec(D, 3), _rows_spec(D, 3), _rows_spec(D),
                 _rows_spec(1), _rows_spec(D), _full_spec((D, D)),
                 _full_spec((D, D)), _full_spec((1, D)), _full_spec((D, D))],
                _rows_spec(D),
                jax.ShapeDtypeStruct((MP, D), _f32))(
        sv2p, ocd, efeat_p, invDE_p, e2, w2a, w2b, b2, W_e)

    return (vfo_p[:N], efo_p[:M])
